# baseline (device time: 8939 ns/iter reference)
import jax
import jax.numpy as jnp
from jax import lax
from jax.experimental import pallas as pl
from jax.experimental.pallas import tpu as pltpu

N_GLOBAL = 1024
EPS = 1e-5


def kernel(x, gamma, beta):
    m, n = x.shape

    def body(x_ref, g_ref, b_ref, out_ref, local_ref, remote_ref, send_sem, recv_sem):
        my_x = lax.axis_index("x")
        my_y = lax.axis_index("y")
        peer = (my_x, 1 - my_y)

        barrier_sem = pltpu.get_barrier_semaphore()
        pl.semaphore_signal(
            barrier_sem, inc=1, device_id=peer, device_id_type=pl.DeviceIdType.MESH
        )

        xv = x_ref[:, :].astype(jnp.float32)
        local_ref[0, :] = jnp.sum(xv, axis=1)
        local_ref[1, :] = jnp.sum(xv * xv, axis=1)

        pl.semaphore_wait(barrier_sem, 1)

        rdma = pltpu.make_async_remote_copy(
            src_ref=local_ref,
            dst_ref=remote_ref,
            send_sem=send_sem,
            recv_sem=recv_sem,
            device_id=peer,
            device_id_type=pl.DeviceIdType.MESH,
        )
        rdma.start()
        rdma.wait_recv()

        total_s = local_ref[0, :] + remote_ref[0, :]
        total_sq = local_ref[1, :] + remote_ref[1, :]
        mean = total_s * (1.0 / N_GLOBAL)
        var = total_sq * (1.0 / N_GLOBAL) - mean * mean
        inv = lax.rsqrt(var + EPS)
        norm = (xv - mean[:, None]) * inv[:, None]
        out_ref[:, :] = (g_ref[:] * norm + b_ref[:]).astype(out_ref.dtype)

        rdma.wait_send()

    return pl.pallas_call(
        body,
        out_shape=jax.ShapeDtypeStruct((m, n), x.dtype),
        in_specs=[
            pl.BlockSpec(memory_space=pltpu.VMEM),
            pl.BlockSpec(memory_space=pltpu.VMEM),
            pl.BlockSpec(memory_space=pltpu.VMEM),
        ],
        out_specs=pl.BlockSpec(memory_space=pltpu.VMEM),
        scratch_shapes=[
            pltpu.VMEM((2, m), jnp.float32),
            pltpu.VMEM((2, m), jnp.float32),
            pltpu.SemaphoreType.DMA,
            pltpu.SemaphoreType.DMA,
        ],
        compiler_params=pltpu.CompilerParams(collective_id=0),
    )(x, gamma, beta)


# device time: 8885 ns/iter; 1.0061x vs baseline; 1.0061x over previous
import jax
import jax.numpy as jnp
from jax import lax
from jax.experimental import pallas as pl
from jax.experimental.pallas import tpu as pltpu

N_GLOBAL = 1024
EPS = 1e-5


def kernel(x, gamma, beta):
    m, n = x.shape

    def body(x_ref, g_ref, b_ref, out_ref, local_ref, remote_ref, send_sem, recv_sem):
        my_x = lax.axis_index("x")
        my_y = lax.axis_index("y")
        peer = (my_x, 1 - my_y)

        barrier_sem = pltpu.get_barrier_semaphore()
        pl.semaphore_signal(
            barrier_sem, inc=2, device_id=peer, device_id_type=pl.DeviceIdType.MESH
        )

        xv = x_ref[:, :].astype(jnp.float32)
        local_ref[0, :] = jnp.sum(xv, axis=1)
        local_ref[1, :] = jnp.sum(xv * xv, axis=1)

        pl.semaphore_wait(barrier_sem, 1)

        rdma = pltpu.make_async_remote_copy(
            src_ref=local_ref,
            dst_ref=remote_ref,
            send_sem=send_sem,
            recv_sem=recv_sem,
            device_id=peer,
            device_id_type=pl.DeviceIdType.MESH,
        )
        rdma.start()
        rdma.wait_recv()

        total_s = local_ref[0, :] + remote_ref[0, :]
        total_sq = local_ref[1, :] + remote_ref[1, :]
        mean = total_s * (1.0 / N_GLOBAL)
        var = total_sq * (1.0 / N_GLOBAL) - mean * mean
        inv = lax.rsqrt(var + EPS)
        norm = (xv - mean[:, None]) * inv[:, None]
        out_ref[:, :] = (g_ref[:] * norm + b_ref[:]).astype(out_ref.dtype)

        rdma.wait_send()

    return pl.pallas_call(
        body,
        out_shape=jax.ShapeDtypeStruct((m, n), x.dtype),
        in_specs=[
            pl.BlockSpec(memory_space=pltpu.VMEM),
            pl.BlockSpec(memory_space=pltpu.VMEM),
            pl.BlockSpec(memory_space=pltpu.VMEM),
        ],
        out_specs=pl.BlockSpec(memory_space=pltpu.VMEM),
        scratch_shapes=[
            pltpu.VMEM((2, m), jnp.float32),
            pltpu.VMEM((2, m), jnp.float32),
            pltpu.SemaphoreType.DMA,
            pltpu.SemaphoreType.DMA,
        ],
        compiler_params=pltpu.CompilerParams(collective_id=0),
    )(x, gamma, beta)
